# baseline (speedup 1.0000x reference)
import jax
import jax.numpy as jnp
from jax.experimental import pallas as pl
from jax.experimental.pallas import tpu as pltpu

_TB = 128


def _fused_ae_kernel(x_ref, gv_ref, g_ref, we_ref, be_ref, wd_ref, bd_ref, out_ref):
    xa = (x_ref[...] + g_ref[...] * gv_ref[...]).astype(jnp.bfloat16)
    h = jnp.dot(xa, we_ref[...], preferred_element_type=jnp.float32)
    h = jnp.maximum(h + be_ref[...], 0.0).astype(jnp.bfloat16)
    out = jnp.dot(h, wd_ref[...], preferred_element_type=jnp.float32)
    out_ref[...] = out + bd_ref[...]


def kernel(x, genre_vec, g, W_enc, b_enc, W_dec, b_dec):
    B, SIZE = x.shape
    HIDDEN = W_enc.shape[1]
    b_enc2 = b_enc.reshape(1, HIDDEN)
    b_dec2 = b_dec.reshape(1, SIZE)
    We16 = W_enc.astype(jnp.bfloat16)
    Wd16 = W_dec.astype(jnp.bfloat16)

    grid = (B // _TB,)
    return pl.pallas_call(
        _fused_ae_kernel,
        grid=grid,
        in_specs=[
            pl.BlockSpec((_TB, SIZE), lambda i: (i, 0)),
            pl.BlockSpec((_TB, SIZE), lambda i: (i, 0)),
            pl.BlockSpec((1, SIZE), lambda i: (0, 0)),
            pl.BlockSpec((SIZE, HIDDEN), lambda i: (0, 0)),
            pl.BlockSpec((1, HIDDEN), lambda i: (0, 0)),
            pl.BlockSpec((HIDDEN, SIZE), lambda i: (0, 0)),
            pl.BlockSpec((1, SIZE), lambda i: (0, 0)),
        ],
        out_specs=pl.BlockSpec((_TB, SIZE), lambda i: (i, 0)),
        out_shape=jax.ShapeDtypeStruct((B, SIZE), jnp.float32),
        compiler_params=pltpu.CompilerParams(
            dimension_semantics=("parallel",),
        ),
    )(x, genre_vec, g, We16, b_enc2, Wd16, b_dec2)


# speedup vs baseline: 1.0508x; 1.0508x over previous
import jax
import jax.numpy as jnp
from jax.experimental import pallas as pl
from jax.experimental.pallas import tpu as pltpu

_TB = 128
_IN_BUFS = 3


def _outer(x_hbm, gv_hbm, g_ref, we_ref, be_ref, wd_ref, bd_ref, out_hbm):
    size = x_hbm.shape[1]
    nsteps = x_hbm.shape[0] // _TB

    def inner(x_blk, gv_blk, out_blk):
        xa = x_blk[...] + g_ref[...] * gv_blk[...]
        h = jnp.dot(xa, we_ref[...], preferred_element_type=jnp.float32)
        h = jnp.maximum(h + be_ref[...], 0.0)
        out = jnp.dot(h, wd_ref[...], preferred_element_type=jnp.float32)
        out_blk[...] = out + bd_ref[...]

    pltpu.emit_pipeline(
        inner,
        grid=(nsteps,),
        in_specs=[
            pl.BlockSpec((_TB, size), lambda i: (i, 0),
                         pipeline_mode=pl.Buffered(buffer_count=_IN_BUFS)),
            pl.BlockSpec((_TB, size), lambda i: (i, 0),
                         pipeline_mode=pl.Buffered(buffer_count=_IN_BUFS)),
        ],
        out_specs=[
            pl.BlockSpec((_TB, size), lambda i: (i, 0),
                         pipeline_mode=pl.Buffered(buffer_count=2)),
        ],
    )(x_hbm, gv_hbm, out_hbm)


def kernel(x, genre_vec, g, W_enc, b_enc, W_dec, b_dec):
    B, SIZE = x.shape
    HIDDEN = W_enc.shape[1]
    b_enc2 = b_enc.reshape(1, HIDDEN)
    b_dec2 = b_dec.reshape(1, SIZE)

    return pl.pallas_call(
        _outer,
        in_specs=[
            pl.BlockSpec(memory_space=pl.ANY),
            pl.BlockSpec(memory_space=pl.ANY),
            pl.BlockSpec((1, SIZE), lambda: (0, 0)),
            pl.BlockSpec((SIZE, HIDDEN), lambda: (0, 0)),
            pl.BlockSpec((1, HIDDEN), lambda: (0, 0)),
            pl.BlockSpec((HIDDEN, SIZE), lambda: (0, 0)),
            pl.BlockSpec((1, SIZE), lambda: (0, 0)),
        ],
        out_specs=pl.BlockSpec(memory_space=pl.ANY),
        out_shape=jax.ShapeDtypeStruct((B, SIZE), jnp.float32),
    )(x, genre_vec, g, W_enc, b_enc2, W_dec, b_dec2)


# speedup vs baseline: 1.0527x; 1.0018x over previous
import jax
import jax.numpy as jnp
from jax.experimental import pallas as pl
from jax.experimental.pallas import tpu as pltpu

_TB = 128
_IN_BUFS = 4


def _outer(x_hbm, gv_hbm, g_ref, we_ref, be_ref, wd_ref, bd_ref, out_hbm):
    size = x_hbm.shape[1]
    nsteps = x_hbm.shape[0] // _TB

    def inner(x_blk, gv_blk, out_blk):
        xa = x_blk[...] + g_ref[...] * gv_blk[...]
        h = jnp.dot(xa, we_ref[...], preferred_element_type=jnp.float32)
        h = jnp.maximum(h + be_ref[...], 0.0)
        out = jnp.dot(h, wd_ref[...], preferred_element_type=jnp.float32)
        out_blk[...] = out + bd_ref[...]

    pltpu.emit_pipeline(
        inner,
        grid=(nsteps,),
        in_specs=[
            pl.BlockSpec((_TB, size), lambda i: (i, 0),
                         pipeline_mode=pl.Buffered(buffer_count=_IN_BUFS)),
            pl.BlockSpec((_TB, size), lambda i: (i, 0),
                         pipeline_mode=pl.Buffered(buffer_count=_IN_BUFS)),
        ],
        out_specs=[
            pl.BlockSpec((_TB, size), lambda i: (i, 0),
                         pipeline_mode=pl.Buffered(buffer_count=2)),
        ],
    )(x_hbm, gv_hbm, out_hbm)


def kernel(x, genre_vec, g, W_enc, b_enc, W_dec, b_dec):
    B, SIZE = x.shape
    HIDDEN = W_enc.shape[1]
    b_enc2 = b_enc.reshape(1, HIDDEN)
    b_dec2 = b_dec.reshape(1, SIZE)

    return pl.pallas_call(
        _outer,
        in_specs=[
            pl.BlockSpec(memory_space=pl.ANY),
            pl.BlockSpec(memory_space=pl.ANY),
            pl.BlockSpec((1, SIZE), lambda: (0, 0)),
            pl.BlockSpec((SIZE, HIDDEN), lambda: (0, 0)),
            pl.BlockSpec((1, HIDDEN), lambda: (0, 0)),
            pl.BlockSpec((HIDDEN, SIZE), lambda: (0, 0)),
            pl.BlockSpec((1, SIZE), lambda: (0, 0)),
        ],
        out_specs=pl.BlockSpec(memory_space=pl.ANY),
        out_shape=jax.ShapeDtypeStruct((B, SIZE), jnp.float32),
    )(x, genre_vec, g, W_enc, b_enc2, W_dec, b_dec2)


# speedup vs baseline: 1.0575x; 1.0046x over previous
import jax
import jax.numpy as jnp
from jax.experimental import pallas as pl
from jax.experimental.pallas import tpu as pltpu

_TB = 64
_IN_BUFS = 6


def _outer(x_hbm, gv_hbm, g_ref, we_ref, be_ref, wd_ref, bd_ref, out_hbm):
    size = x_hbm.shape[1]
    nsteps = x_hbm.shape[0] // _TB

    def inner(x_blk, gv_blk, out_blk):
        xa = x_blk[...] + g_ref[...] * gv_blk[...]
        h = jnp.dot(xa, we_ref[...], preferred_element_type=jnp.float32)
        h = jnp.maximum(h + be_ref[...], 0.0)
        out = jnp.dot(h, wd_ref[...], preferred_element_type=jnp.float32)
        out_blk[...] = out + bd_ref[...]

    pltpu.emit_pipeline(
        inner,
        grid=(nsteps,),
        in_specs=[
            pl.BlockSpec((_TB, size), lambda i: (i, 0),
                         pipeline_mode=pl.Buffered(buffer_count=_IN_BUFS)),
            pl.BlockSpec((_TB, size), lambda i: (i, 0),
                         pipeline_mode=pl.Buffered(buffer_count=_IN_BUFS)),
        ],
        out_specs=[
            pl.BlockSpec((_TB, size), lambda i: (i, 0),
                         pipeline_mode=pl.Buffered(buffer_count=2)),
        ],
    )(x_hbm, gv_hbm, out_hbm)


def kernel(x, genre_vec, g, W_enc, b_enc, W_dec, b_dec):
    B, SIZE = x.shape
    HIDDEN = W_enc.shape[1]
    b_enc2 = b_enc.reshape(1, HIDDEN)
    b_dec2 = b_dec.reshape(1, SIZE)

    return pl.pallas_call(
        _outer,
        in_specs=[
            pl.BlockSpec(memory_space=pl.ANY),
            pl.BlockSpec(memory_space=pl.ANY),
            pl.BlockSpec((1, SIZE), lambda: (0, 0)),
            pl.BlockSpec((SIZE, HIDDEN), lambda: (0, 0)),
            pl.BlockSpec((1, HIDDEN), lambda: (0, 0)),
            pl.BlockSpec((HIDDEN, SIZE), lambda: (0, 0)),
            pl.BlockSpec((1, SIZE), lambda: (0, 0)),
        ],
        out_specs=pl.BlockSpec(memory_space=pl.ANY),
        out_shape=jax.ShapeDtypeStruct((B, SIZE), jnp.float32),
    )(x, genre_vec, g, W_enc, b_enc2, W_dec, b_dec2)
